# Initial kernel scaffold; baseline (speedup 1.0000x reference)
import jax
import jax.numpy as jnp
from jax.experimental import pallas as pl


def kernel(x, gate_w, W1, W2, W3):
    raise NotImplementedError("write your pallas kernel here")


# speedup vs baseline: 1.5068x; 1.5068x over previous
import functools
import jax
import jax.numpy as jnp
from jax import lax
from jax.experimental import pallas as pl
from jax.experimental.pallas import tpu as pltpu

DIM = 1024
HIDDEN = 2816
N_EXPERTS = 8
TOP_K = 2
AUX_WEIGHT = 0.01

BH = 256
NH = HIDDEN // BH
BG = 512


def _gate_kernel(x_ref, gw_ref, wts_ref, aux_ref, psum_s, fsum_s):
    m = pl.program_id(0)
    nm = pl.num_programs(0)
    n = x_ref.shape[0]

    logits = jnp.dot(x_ref[...], gw_ref[...].T,
                     preferred_element_type=jnp.float32)
    iota = lax.broadcasted_iota(jnp.int32, (n, N_EXPERTS), 1)
    m1 = jnp.max(logits, axis=1, keepdims=True)
    i1 = jnp.min(jnp.where(logits == m1, iota, N_EXPERTS), axis=1,
                 keepdims=True)
    l2 = jnp.where(iota == i1, -jnp.inf, logits)
    m2 = jnp.max(l2, axis=1, keepdims=True)
    i2 = jnp.min(jnp.where(l2 == m2, iota, N_EXPERTS), axis=1,
                 keepdims=True)
    t = jnp.exp(m2 - m1)
    w_first = 1.0 / (1.0 + t)
    w_second = t / (1.0 + t)
    wts_ref[...] = (jnp.where(iota == i1, w_first, 0.0)
                    + jnp.where(iota == i2, w_second, 0.0))

    pe = jnp.exp(logits - m1)
    probs = pe / jnp.sum(pe, axis=1, keepdims=True)
    psum = jnp.sum(probs, axis=0, keepdims=True)
    fsum = jnp.sum((iota == i1).astype(jnp.float32), axis=0, keepdims=True)

    @pl.when(m == 0)
    def _init():
        psum_s[...] = psum
        fsum_s[...] = fsum

    @pl.when(m != 0)
    def _acc():
        psum_s[...] += psum
        fsum_s[...] += fsum

    @pl.when(m == nm - 1)
    def _aux():
        ntok = jnp.float32(n) * nm
        aux_ref[...] = (AUX_WEIGHT * N_EXPERTS
                        * jnp.sum(psum_s[...] * fsum_s[...], keepdims=True
                                  ).reshape(1, 1) / (ntok * ntok))


def _ffn_kernel(x_ref, wts_ref, w1_ref, w3_ref, w2_ref, out_ref):
    e = pl.program_id(0)
    h = pl.program_id(1)

    onehot = (lax.broadcasted_iota(jnp.int32, (N_EXPERTS, 1), 0)
              == e).astype(jnp.float32)
    wvec = jnp.dot(wts_ref[...], onehot,
                   preferred_element_type=jnp.float32)

    xl = x_ref[...]
    hblk = (jax.nn.silu(jnp.dot(xl, w1_ref[0].T,
                                preferred_element_type=jnp.float32))
            * jnp.dot(xl, w3_ref[0].T, preferred_element_type=jnp.float32))
    contrib = jnp.dot(wvec * hblk, w2_ref[0].T,
                      preferred_element_type=jnp.float32)

    @pl.when((e == 0) & (h == 0))
    def _init():
        out_ref[...] = contrib

    @pl.when(~((e == 0) & (h == 0)))
    def _acc():
        out_ref[...] += contrib


def kernel(x, gate_w, W1, W2, W3):
    B, T, C = x.shape
    N = B * T
    x_flat = x.reshape(N, C)

    wts, aux = pl.pallas_call(
        _gate_kernel,
        grid=(N // BG,),
        in_specs=[
            pl.BlockSpec((BG, C), lambda m: (m, 0)),
            pl.BlockSpec((N_EXPERTS, C), lambda m: (0, 0)),
        ],
        out_specs=[
            pl.BlockSpec((BG, N_EXPERTS), lambda m: (m, 0)),
            pl.BlockSpec((1, 1), lambda m: (0, 0)),
        ],
        out_shape=[
            jax.ShapeDtypeStruct((N, N_EXPERTS), jnp.float32),
            jax.ShapeDtypeStruct((1, 1), jnp.float32),
        ],
        scratch_shapes=[
            pltpu.VMEM((1, N_EXPERTS), jnp.float32),
            pltpu.VMEM((1, N_EXPERTS), jnp.float32),
        ],
    )(x_flat, gate_w)

    out = pl.pallas_call(
        _ffn_kernel,
        grid=(N_EXPERTS, NH),
        in_specs=[
            pl.BlockSpec((N, C), lambda e, h: (0, 0)),
            pl.BlockSpec((N, N_EXPERTS), lambda e, h: (0, 0)),
            pl.BlockSpec((1, BH, C), lambda e, h: (e, h, 0)),
            pl.BlockSpec((1, BH, C), lambda e, h: (e, h, 0)),
            pl.BlockSpec((1, C, BH), lambda e, h: (e, 0, h)),
        ],
        out_specs=pl.BlockSpec((N, C), lambda e, h: (0, 0)),
        out_shape=jax.ShapeDtypeStruct((N, C), jnp.float32),
    )(x_flat, wts, W1, W3, W2)

    return out.reshape(B, T, C), aux.reshape(())[()]


# speedup vs baseline: 1.7861x; 1.1854x over previous
import functools
import jax
import jax.numpy as jnp
from jax import lax
from jax.experimental import pallas as pl
from jax.experimental.pallas import tpu as pltpu
from jax.experimental.pallas import tpu_sc as plsc

DIM = 1024
HIDDEN = 2816
HHALF = HIDDEN // 2
N_EXPERTS = 8
AUX_WEIGHT = 0.01

N = 4096
BG = 512
BM = 128
NM = N // BG
SMAX = 2 * N + 896
NT = SMAX // BM

NW = 32
TPW = N // NW
CH_S = 32
NCH_S = TPW // CH_S
CH_G = 16
NCH_G = TPW // CH_G


def _top2(logits, n):
    iota = lax.broadcasted_iota(jnp.int32, (n, N_EXPERTS), 1)
    m1 = jnp.max(logits, axis=1, keepdims=True)
    i1 = jnp.min(jnp.where(logits == m1, iota, N_EXPERTS), axis=1,
                 keepdims=True)
    l2 = jnp.where(iota == i1, -jnp.inf, logits)
    m2 = jnp.max(l2, axis=1, keepdims=True)
    i2 = jnp.min(jnp.where(l2 == m2, iota, N_EXPERTS), axis=1,
                 keepdims=True)
    oh1 = (iota == i1).astype(jnp.float32)
    oh2 = (iota == i2).astype(jnp.float32)
    return m1, m2, oh1, oh2


def _gate_a_kernel(x_ref, gw_ref, cnt_ref, offp_ref, aux_ref,
                   psum_s, fsum_s, cnt_s):
    m = pl.program_id(0)
    n = x_ref.shape[0]
    logits = jnp.dot(x_ref[...], gw_ref[...].T,
                     preferred_element_type=jnp.float32)
    m1, m2, oh1, oh2 = _top2(logits, n)

    pe = jnp.exp(logits - m1)
    probs = pe / jnp.sum(pe, axis=1, keepdims=True)
    psum = jnp.sum(probs, axis=0, keepdims=True)
    fsum = jnp.sum(oh1, axis=0, keepdims=True)
    csum = jnp.sum(oh1 + oh2, axis=0, keepdims=True)

    @pl.when(m == 0)
    def _init():
        psum_s[...] = psum
        fsum_s[...] = fsum
        cnt_s[...] = csum

    @pl.when(m != 0)
    def _acc():
        psum_s[...] += psum
        fsum_s[...] += fsum
        cnt_s[...] += csum

    @pl.when(m == NM - 1)
    def _fin():
        padded = jnp.floor((cnt_s[...] + (BM - 1)) / BM) * BM
        ei = lax.broadcasted_iota(jnp.int32, (N_EXPERTS, N_EXPERTS), 0)
        ej = lax.broadcasted_iota(jnp.int32, (N_EXPERTS, N_EXPERTS), 1)
        slt = (ei < ej).astype(jnp.float32)
        offp = jnp.dot(padded, slt, preferred_element_type=jnp.float32)
        cnt_ref[...] = cnt_s[...].astype(jnp.int32)
        offp_ref[...] = offp.astype(jnp.int32)
        ntok = jnp.float32(N)
        aux_ref[...] = (AUX_WEIGHT * N_EXPERTS
                        * jnp.sum(psum_s[...] * fsum_s[...],
                                  keepdims=True).reshape(1, 1)
                        / (ntok * ntok))


def _gate_b_kernel(x_ref, gw_ref, offp_ref,
                   pos0_ref, pos1_ref, w0_ref, w1_ref, carry_s):
    m = pl.program_id(0)
    n = x_ref.shape[0]
    logits = jnp.dot(x_ref[...], gw_ref[...].T,
                     preferred_element_type=jnp.float32)
    m1, m2, oh1, oh2 = _top2(logits, n)

    t = jnp.exp(m2 - m1)
    w0_ref[...] = 1.0 / (1.0 + t)
    w1_ref[...] = t / (1.0 + t)

    @pl.when(m == 0)
    def _init():
        carry_s[...] = jnp.zeros_like(carry_s)

    o_all = jnp.concatenate([oh1, oh2], axis=0)
    ri = lax.broadcasted_iota(jnp.int32, (2 * n, 2 * n), 0)
    rj = lax.broadcasted_iota(jnp.int32, (2 * n, 2 * n), 1)
    tri = (ri >= rj).astype(jnp.float32)
    csum = jnp.dot(tri, o_all, preferred_element_type=jnp.float32)
    mtx = csum + carry_s[...] + offp_ref[...].astype(jnp.float32) - 1.0
    posall = jnp.sum(mtx * o_all, axis=1, keepdims=True)
    pos0_ref[...] = posall[:n].astype(jnp.int32)
    pos1_ref[...] = posall[n:].astype(jnp.int32)
    carry_s[...] += jnp.sum(o_all, axis=0, keepdims=True)


def _gate(x_flat, gate_w):
    cnt, offp, aux = pl.pallas_call(
        _gate_a_kernel,
        grid=(NM,),
        in_specs=[
            pl.BlockSpec((BG, DIM), lambda m: (m, 0)),
            pl.BlockSpec((N_EXPERTS, DIM), lambda m: (0, 0)),
        ],
        out_specs=[
            pl.BlockSpec((1, N_EXPERTS), lambda m: (0, 0)),
            pl.BlockSpec((1, N_EXPERTS), lambda m: (0, 0)),
            pl.BlockSpec((1, 1), lambda m: (0, 0)),
        ],
        out_shape=[
            jax.ShapeDtypeStruct((1, N_EXPERTS), jnp.int32),
            jax.ShapeDtypeStruct((1, N_EXPERTS), jnp.int32),
            jax.ShapeDtypeStruct((1, 1), jnp.float32),
        ],
        scratch_shapes=[pltpu.VMEM((1, N_EXPERTS), jnp.float32)] * 3,
    )(x_flat, gate_w)

    pos0, pos1, w0, w1 = pl.pallas_call(
        _gate_b_kernel,
        grid=(NM,),
        in_specs=[
            pl.BlockSpec((BG, DIM), lambda m: (m, 0)),
            pl.BlockSpec((N_EXPERTS, DIM), lambda m: (0, 0)),
            pl.BlockSpec((1, N_EXPERTS), lambda m: (0, 0)),
        ],
        out_specs=[
            pl.BlockSpec((BG, 1), lambda m: (m, 0)),
            pl.BlockSpec((BG, 1), lambda m: (m, 0)),
            pl.BlockSpec((BG, 1), lambda m: (m, 0)),
            pl.BlockSpec((BG, 1), lambda m: (m, 0)),
        ],
        out_shape=[
            jax.ShapeDtypeStruct((N, 1), jnp.int32),
            jax.ShapeDtypeStruct((N, 1), jnp.int32),
            jax.ShapeDtypeStruct((N, 1), jnp.float32),
            jax.ShapeDtypeStruct((N, 1), jnp.float32),
        ],
        scratch_shapes=[pltpu.VMEM((1, N_EXPERTS), jnp.float32)],
    )(x_flat, gate_w, offp)
    return pos0, pos1, w0, w1, cnt, offp, aux


_sc_mesh = plsc.VectorSubcoreMesh(core_axis_name="c", subcore_axis_name="s")


@functools.partial(
    pl.kernel,
    mesh=_sc_mesh,
    out_type=jax.ShapeDtypeStruct((SMAX, DIM), jnp.float32),
    scratch_types=[
        pltpu.VMEM((NCH_S, CH_S), jnp.int32),
        pltpu.VMEM((NCH_S, CH_S), jnp.int32),
        pltpu.VMEM((CH_S, DIM), jnp.float32),
        pltpu.SemaphoreType.DMA,
        pltpu.SemaphoreType.DMA,
    ],
)
def _sc_dispatch(x_hbm, pos0_hbm, pos1_hbm, xs_hbm,
                 p0_v, p1_v, xbuf, sem0, sem1):
    wid = lax.axis_index("s") * 2 + lax.axis_index("c")
    pltpu.sync_copy(pos0_hbm.at[wid], p0_v)
    pltpu.sync_copy(pos1_hbm.at[wid], p1_v)
    base = wid * TPW
    for c in range(NCH_S):
        pltpu.sync_copy(x_hbm.at[pl.ds(base + c * CH_S, CH_S)], xbuf)
        cp0 = pltpu.async_copy(xbuf, xs_hbm.at[p0_v.at[c]], sem0)
        cp1 = pltpu.async_copy(xbuf, xs_hbm.at[p1_v.at[c]], sem1)
        cp0.wait()
        cp1.wait()


def _ffn_kernel(te_ref, tm_ref, xs_ref, w1_ref, w3_ref, w2_ref, ys_ref):
    xl = xs_ref[...]
    hb = (jax.nn.silu(jnp.dot(xl, w1_ref[0].T,
                              preferred_element_type=jnp.float32))
          * jnp.dot(xl, w3_ref[0].T, preferred_element_type=jnp.float32))
    ys_ref[0] = jnp.dot(hb, w2_ref[0].T, preferred_element_type=jnp.float32)


def _ffn(xs, W1, W3, W2, tile_e, tile_m):
    grid_spec = pltpu.PrefetchScalarGridSpec(
        num_scalar_prefetch=2,
        grid=(2, NT),
        in_specs=[
            pl.BlockSpec((BM, DIM), lambda h, i, te, tm: (tm[i], 0)),
            pl.BlockSpec((1, HHALF, DIM), lambda h, i, te, tm: (te[i], h, 0)),
            pl.BlockSpec((1, HHALF, DIM), lambda h, i, te, tm: (te[i], h, 0)),
            pl.BlockSpec((1, DIM, HHALF), lambda h, i, te, tm: (te[i], 0, h)),
        ],
        out_specs=pl.BlockSpec((1, BM, DIM),
                               lambda h, i, te, tm: (h, tm[i], 0)),
    )
    return pl.pallas_call(
        _ffn_kernel,
        grid_spec=grid_spec,
        out_shape=jax.ShapeDtypeStruct((2, SMAX, DIM), jnp.float32),
    )(tile_e, tile_m, xs, W1, W3, W2)


def _tile_maps(cnt, offp):
    t_e = (cnt + BM - 1) // BM
    st = offp // BM
    ends = st + t_e
    nta = jnp.sum(t_e)
    i = jnp.arange(NT, dtype=jnp.int32)
    e_i = jnp.sum((i[:, None] >= ends[None, :]).astype(jnp.int32), axis=1)
    tile_e = jnp.minimum(e_i, N_EXPERTS - 1)
    tile_m = jnp.where(i < nta, i, NT - 1)
    return tile_e, tile_m


@functools.partial(
    pl.kernel,
    mesh=_sc_mesh,
    out_type=jax.ShapeDtypeStruct((4, N, DIM), jnp.float32),
    scratch_types=(
        [pltpu.VMEM((NCH_G, CH_G), jnp.int32)] * 4
        + [pltpu.VMEM((CH_G, DIM), jnp.float32)] * 4
        + [pltpu.SemaphoreType.DMA] * 4
    ),
)
def _sc_gather(ysf_hbm, ia_hbm, ib_hbm, ic_hbm, id_hbm, g_hbm,
               iav, ibv, icv, idv, bufa, bufb, bufc, bufd,
               sa, sb, sc, sd):
    wid = lax.axis_index("s") * 2 + lax.axis_index("c")
    pltpu.sync_copy(ia_hbm.at[wid], iav)
    pltpu.sync_copy(ib_hbm.at[wid], ibv)
    pltpu.sync_copy(ic_hbm.at[wid], icv)
    pltpu.sync_copy(id_hbm.at[wid], idv)
    base = wid * TPW
    for c in range(NCH_G):
        ca = pltpu.async_copy(ysf_hbm.at[iav.at[c]], bufa, sa)
        cb = pltpu.async_copy(ysf_hbm.at[ibv.at[c]], bufb, sb)
        cc = pltpu.async_copy(ysf_hbm.at[icv.at[c]], bufc, sc)
        cd = pltpu.async_copy(ysf_hbm.at[idv.at[c]], bufd, sd)
        ca.wait()
        cb.wait()
        cc.wait()
        cd.wait()
        row0 = base + c * CH_G
        pltpu.sync_copy(bufa, g_hbm.at[0].at[pl.ds(row0, CH_G)])
        pltpu.sync_copy(bufb, g_hbm.at[1].at[pl.ds(row0, CH_G)])
        pltpu.sync_copy(bufc, g_hbm.at[2].at[pl.ds(row0, CH_G)])
        pltpu.sync_copy(bufd, g_hbm.at[3].at[pl.ds(row0, CH_G)])


def _combine_kernel(g0_ref, g1_ref, g2_ref, g3_ref, w0_ref, w1_ref, out_ref):
    out_ref[...] = (w0_ref[...] * (g0_ref[0] + g1_ref[0])
                    + w1_ref[...] * (g2_ref[0] + g3_ref[0]))


def _combine(g, w0, w1):
    return pl.pallas_call(
        _combine_kernel,
        grid=(NM,),
        in_specs=[
            pl.BlockSpec((1, BG, DIM), lambda m: (0, m, 0)),
            pl.BlockSpec((1, BG, DIM), lambda m: (1, m, 0)),
            pl.BlockSpec((1, BG, DIM), lambda m: (2, m, 0)),
            pl.BlockSpec((1, BG, DIM), lambda m: (3, m, 0)),
            pl.BlockSpec((BG, 1), lambda m: (m, 0)),
            pl.BlockSpec((BG, 1), lambda m: (m, 0)),
        ],
        out_specs=pl.BlockSpec((BG, DIM), lambda m: (m, 0)),
        out_shape=jax.ShapeDtypeStruct((N, DIM), jnp.float32),
    )(g, g, g, g, w0, w1)


def kernel(x, gate_w, W1, W2, W3):
    B, T, C = x.shape
    x_flat = x.reshape(N, C)

    pos0, pos1, w0, w1, cnt, offp, aux = _gate(x_flat, gate_w)
    tile_e, tile_m = _tile_maps(cnt[0], offp[0])

    p0r = pos0.reshape(NW, NCH_S, CH_S)
    p1r = pos1.reshape(NW, NCH_S, CH_S)
    xs = _sc_dispatch(x_flat, p0r, p1r)

    ys = _ffn(xs, W1, W3, W2, tile_e, tile_m)
    ysf = ys.reshape(2 * SMAX, DIM)

    ia = pos0.reshape(NW, NCH_G, CH_G)
    ib = ia + SMAX
    ic = pos1.reshape(NW, NCH_G, CH_G)
    idx = ic + SMAX
    g = _sc_gather(ysf, ia, ib, ic, idx)

    out = _combine(g, w0, w1)
    return out.reshape(B, T, C), aux.reshape(())[()]
